# baseline (speedup 1.0000x reference)
import functools

import jax
import jax.numpy as jnp
from jax import lax
from jax.experimental import pallas as pl
from jax.experimental.pallas import tpu as pltpu
from jax.experimental.pallas import tpu_sc as plsc

C = 128
N = 10000
K = 32
NC, NS = 2, 16
NW = NC * NS
T_F = C // NW
CH_N = 400
CH_NG = CH_N // 16
N_CHUNK = N // CH_N

TC_BLK = N
TC_GRID = 1


def _h_body(x_ref, w1_ref, b1_ref, h_ref):
    h = lax.dot_general(w1_ref[...], x_ref[...],
                        dimension_numbers=(((1,), (0,)), ((), ())),
                        preferred_element_type=jnp.float32)
    h_ref[...] = jnp.maximum(h + b1_ref[...], 0.0)


def _compute_h(x_cn, w1, b1):
    return pl.pallas_call(
        _h_body,
        grid=(TC_GRID,),
        in_specs=[
            pl.BlockSpec((C, TC_BLK), lambda i: (0, i)),
            pl.BlockSpec((C, C), lambda i: (0, 0)),
            pl.BlockSpec((C, 1), lambda i: (0, 0)),
        ],
        out_specs=pl.BlockSpec((C, TC_BLK), lambda i: (0, i)),
        out_shape=jax.ShapeDtypeStruct((C, N), jnp.float32),
    )(x_cn, w1, b1.reshape(C, 1))


@functools.cache
def _make_sc_gather_max():
    @functools.partial(
        pl.kernel,
        out_type=jax.ShapeDtypeStruct((NW, T_F, N_CHUNK, CH_NG, 16),
                                      jnp.float32),
        mesh=plsc.VectorSubcoreMesh(core_axis_name="c", subcore_axis_name="s"),
        compiler_params=pltpu.CompilerParams(needs_layout_passes=False,
                                             use_tc_tiling_on_sc=False),
        scratch_types=[
            pltpu.VMEM((T_F * N,), jnp.float32),
            pltpu.VMEM((2, CH_N * K), jnp.int32),
            pltpu.VMEM((2, T_F, CH_NG, 16), jnp.float32),
            pltpu.SemaphoreType.DMA,
            pltpu.SemaphoreType.DMA,
            pltpu.SemaphoreType.DMA,
            pltpu.SemaphoreType.DMA,
        ],
    )
    def _sc_gather_max(h, idxs, out, table_v, idx_ring, out_ring,
                       sem_i0, sem_i1, sem_o0, sem_o1):
        t = lax.axis_index("s") * NC + lax.axis_index("c")
        sems_i = [sem_i0, sem_i1]
        sems_o = [sem_o0, sem_o1]

        pltpu.sync_copy(h.at[pl.ds(t * (T_F * N), T_F * N)], table_v)

        lanes_k = lax.iota(jnp.int32, 16) * K

        def start_idx(c, rb):
            pltpu.async_copy(idxs.at[pl.ds(c * (CH_N * K), CH_N * K)],
                             idx_ring.at[rb], sems_i[rb])

        def drain_idx(rb):
            pltpu.make_async_copy(idxs.at[pl.ds(0, CH_N * K)],
                                  idx_ring.at[rb], sems_i[rb]).wait()

        def start_out(c, rb):
            for f in range(T_F):
                pltpu.async_copy(out_ring.at[rb, f], out.at[t, f, c],
                                 sems_o[rb])

        def drain_out(rb):
            for f in range(T_F):
                pltpu.make_async_copy(out_ring.at[rb, f], out.at[t, f, 0],
                                      sems_o[rb]).wait()

        start_idx(0, 0)

        def chunk_body(c, rb):
            @pl.when(c + 1 < N_CHUNK)
            def _():
                start_idx(c + 1, 1 - rb)

            drain_idx(rb)

            @pl.when(c >= 2)
            def _():
                drain_out(rb)

            iring = idx_ring.at[rb]

            def ng_body(ng, _):
                base = ng * (16 * K)
                accs = None
                for k in range(K):
                    nbr = plsc.load_gather(iring, [lanes_k + (base + k)])
                    vals = [plsc.load_gather(table_v, [nbr + (f * N)])
                            if f else plsc.load_gather(table_v, [nbr])
                            for f in range(T_F)]
                    if accs is None:
                        accs = vals
                    else:
                        accs = [jnp.maximum(a, v) for a, v in zip(accs, vals)]
                for f in range(T_F):
                    out_ring[rb, f, ng, :] = accs[f]
                return 0

            lax.fori_loop(0, CH_NG, ng_body, 0)
            start_out(c, rb)

        def chunk_pair(c2, _):
            for rb in range(2):
                c = 2 * c2 + rb

                @pl.when(c < N_CHUNK)
                def _():
                    chunk_body(c, rb)
            return 0

        lax.fori_loop(0, (N_CHUNK + 1) // 2, chunk_pair, 0)
        drain_out(0)
        drain_out(1)

    return _sc_gather_max


def _out_body(x_ref, m_ref, w2a_ref, w2b_ref, b2_ref, o_ref):
    a = lax.dot_general(w2a_ref[...], x_ref[...],
                        dimension_numbers=(((1,), (0,)), ((), ())),
                        preferred_element_type=jnp.float32)
    b = lax.dot_general(w2b_ref[...], m_ref[...],
                        dimension_numbers=(((1,), (0,)), ((), ())),
                        preferred_element_type=jnp.float32)
    o_ref[...] = jnp.maximum(a + b + b2_ref[...], 0.0)


def _compute_out(x_cn, m_cn, w2a, w2b, b2):
    return pl.pallas_call(
        _out_body,
        grid=(TC_GRID,),
        in_specs=[
            pl.BlockSpec((C, TC_BLK), lambda i: (0, i)),
            pl.BlockSpec((C, TC_BLK), lambda i: (0, i)),
            pl.BlockSpec((C, C), lambda i: (0, 0)),
            pl.BlockSpec((C, C), lambda i: (0, 0)),
            pl.BlockSpec((C, 1), lambda i: (0, 0)),
        ],
        out_specs=pl.BlockSpec((C, TC_BLK), lambda i: (0, i)),
        out_shape=jax.ShapeDtypeStruct((C, N), jnp.float32),
    )(x_cn, m_cn, w2a, w2b, b2.reshape(C, 1))


def kernel(x, edge_index, W1, b1, W2, b2):
    x_cn = x[0, :, :, 0]
    idx_flat = edge_index[0, 0].astype(jnp.int32).reshape(N * K)

    h = _compute_h(x_cn, W1, b1)
    m_raw = _make_sc_gather_max()(h.reshape(C * N), idx_flat)
    m_cn = m_raw.reshape(C, N)
    out_cn = _compute_out(x_cn, m_cn, W2[:, :C], W2[:, C:], b2)
    return out_cn.reshape(1, C, N, 1)


# speedup vs baseline: 1.9064x; 1.9064x over previous
import functools

import jax
import jax.numpy as jnp
from jax import lax
from jax.experimental import pallas as pl
from jax.experimental.pallas import tpu as pltpu
from jax.experimental.pallas import tpu_sc as plsc

C = 128
N = 10000
K = 32
NC, NS = 2, 16
NW = NC * NS
T_F = C // NW
CH_N = 400
CH_NG = CH_N // 16
N_CHUNK = N // CH_N

TC_BLK = N
TC_GRID = 1


def _h_body(x_ref, w1_ref, b1_ref, h_ref):
    h = lax.dot_general(w1_ref[...], x_ref[...],
                        dimension_numbers=(((1,), (0,)), ((), ())),
                        preferred_element_type=jnp.float32)
    h_ref[...] = jnp.maximum(h + b1_ref[...], 0.0)


def _compute_h(x_cn, w1, b1):
    return pl.pallas_call(
        _h_body,
        grid=(TC_GRID,),
        in_specs=[
            pl.BlockSpec((C, TC_BLK), lambda i: (0, i)),
            pl.BlockSpec((C, C), lambda i: (0, 0)),
            pl.BlockSpec((C, 1), lambda i: (0, 0)),
        ],
        out_specs=pl.BlockSpec((C, TC_BLK), lambda i: (0, i)),
        out_shape=jax.ShapeDtypeStruct((C, N), jnp.float32),
    )(x_cn, w1, b1.reshape(C, 1))


@functools.cache
def _make_sc_gather_max():
    @functools.partial(
        pl.kernel,
        out_type=jax.ShapeDtypeStruct((NW, T_F, N_CHUNK, CH_NG, 16),
                                      jnp.float32),
        mesh=plsc.VectorSubcoreMesh(core_axis_name="c", subcore_axis_name="s"),
        compiler_params=pltpu.CompilerParams(needs_layout_passes=False,
                                             use_tc_tiling_on_sc=False),
        scratch_types=[
            pltpu.VMEM((T_F * N,), jnp.float32),
            pltpu.VMEM((2, CH_NG, K, 16), jnp.int32),
            pltpu.VMEM((2, T_F, CH_NG, 16), jnp.float32),
            pltpu.SemaphoreType.DMA,
            pltpu.SemaphoreType.DMA,
            pltpu.SemaphoreType.DMA,
            pltpu.SemaphoreType.DMA,
        ],
    )
    def _sc_gather_max(h, idx4, out, table_v, idx_ring, out_ring,
                       sem_i0, sem_i1, sem_o0, sem_o1):
        t = lax.axis_index("s") * NC + lax.axis_index("c")
        sems_i = [sem_i0, sem_i1]
        sems_o = [sem_o0, sem_o1]

        pltpu.sync_copy(h.at[pl.ds(t * (T_F * N), T_F * N)], table_v)

        def start_idx(c, rb):
            pltpu.async_copy(idx4.at[c], idx_ring.at[rb], sems_i[rb])

        def drain_idx(rb):
            pltpu.make_async_copy(idx4.at[0], idx_ring.at[rb],
                                  sems_i[rb]).wait()

        def start_out(c, rb):
            for f in range(T_F):
                pltpu.async_copy(out_ring.at[rb, f], out.at[t, f, c],
                                 sems_o[rb])

        def drain_out(rb):
            for f in range(T_F):
                pltpu.make_async_copy(out_ring.at[rb, f], out.at[t, f, 0],
                                      sems_o[rb]).wait()

        start_idx(0, 0)

        def chunk_body(c, rb):
            @pl.when(c + 1 < N_CHUNK)
            def _():
                start_idx(c + 1, 1 - rb)

            drain_idx(rb)

            @pl.when(c >= 2)
            def _():
                drain_out(rb)

            def ng_body(ng, _):
                ivs = [idx_ring[rb, ng, k, :] for k in range(K)]
                for f in range(T_F):
                    fvs = [iv + (f * N) for iv in ivs] if f else ivs
                    acc = plsc.load_gather(table_v, [fvs[0]])
                    for k in range(1, K):
                        acc = jnp.maximum(
                            acc, plsc.load_gather(table_v, [fvs[k]]))
                    out_ring[rb, f, ng, :] = acc
                return 0

            lax.fori_loop(0, CH_NG, ng_body, 0)
            start_out(c, rb)

        def chunk_pair(c2, _):
            for rb in range(2):
                c = 2 * c2 + rb

                @pl.when(c < N_CHUNK)
                def _():
                    chunk_body(c, rb)
            return 0

        lax.fori_loop(0, (N_CHUNK + 1) // 2, chunk_pair, 0)
        drain_out(0)
        drain_out(1)

    return _sc_gather_max


def _out_body(x_ref, m_ref, w2a_ref, w2b_ref, b2_ref, o_ref):
    a = lax.dot_general(w2a_ref[...], x_ref[...],
                        dimension_numbers=(((1,), (0,)), ((), ())),
                        preferred_element_type=jnp.float32)
    b = lax.dot_general(w2b_ref[...], m_ref[...],
                        dimension_numbers=(((1,), (0,)), ((), ())),
                        preferred_element_type=jnp.float32)
    o_ref[...] = jnp.maximum(a + b + b2_ref[...], 0.0)


def _compute_out(x_cn, m_cn, w2a, w2b, b2):
    return pl.pallas_call(
        _out_body,
        grid=(TC_GRID,),
        in_specs=[
            pl.BlockSpec((C, TC_BLK), lambda i: (0, i)),
            pl.BlockSpec((C, TC_BLK), lambda i: (0, i)),
            pl.BlockSpec((C, C), lambda i: (0, 0)),
            pl.BlockSpec((C, C), lambda i: (0, 0)),
            pl.BlockSpec((C, 1), lambda i: (0, 0)),
        ],
        out_specs=pl.BlockSpec((C, TC_BLK), lambda i: (0, i)),
        out_shape=jax.ShapeDtypeStruct((C, N), jnp.float32),
    )(x_cn, m_cn, w2a, w2b, b2.reshape(C, 1))


def kernel(x, edge_index, W1, b1, W2, b2):
    x_cn = x[0, :, :, 0]
    idx = edge_index[0, 0].astype(jnp.int32)
    idx4 = idx.reshape(N_CHUNK, CH_NG, 16, K).transpose(0, 1, 3, 2)

    h = _compute_h(x_cn, W1, b1)
    m_raw = _make_sc_gather_max()(h.reshape(C * N), idx4)
    m_cn = m_raw.reshape(C, N)
    out_cn = _compute_out(x_cn, m_cn, W2[:, :C], W2[:, C:], b2)
    return out_cn.reshape(1, C, N, 1)
